# baseline (speedup 1.0000x reference)
import jax
import jax.numpy as jnp
import numpy as np
from jax import lax
from jax.experimental import pallas as pl
from jax.experimental.pallas import tpu as pltpu
from jax.experimental.pallas import tpu_sc as plsc

B = 16384
I = 100
IP = 112
V = 31
E = 64
NC = 2
NS = 16
EH = E // NC
TROWS = I * V + 4
RPT = B // NS
RC = 256
NCHUNK = RPT // RC

_HIMASK = np.int32(-65536)


def _sc_kernel(x_hbm, tab_hbm, out_hbm, tab_v, x_v, out_v,
               sx0, sx1, so0, so1):
    sx = (sx0, sx1)
    so = (so0, so1)
    c = lax.axis_index("c")
    s = lax.axis_index("s")

    def start_x(buf, k):
        pltpu.async_copy(
            x_hbm.at[pl.ds(s * RPT + k * RC, RC), :], x_v.at[buf], sx[buf]
        )

    def wait_x(buf):
        pltpu.make_async_copy(
            x_hbm.at[pl.ds(0, RC), :], x_v.at[buf], sx[buf]
        ).wait()

    def start_o(buf, k):
        pltpu.async_copy(
            out_v.at[buf], out_hbm.at[c, pl.ds(s * RPT + k * RC, RC), :],
            so[buf],
        )

    def wait_o(buf):
        pltpu.make_async_copy(
            out_v.at[buf], out_hbm.at[c, pl.ds(0, RC), :], so[buf]
        ).wait()

    ht = pltpu.async_copy(tab_hbm.at[c], tab_v, sx[1])
    start_x(0, 0)
    ht.wait()

    def chunk_body(k2, _):
        for par in range(2):
            k = 2 * k2 + par

            @pl.when(k + 1 < NCHUNK)
            def _():
                start_x(1 - par, k + 1)

            wait_x(par)

            @pl.when(k >= 2)
            def _():
                wait_o(par)

            _compute_chunk(tab_v, x_v, out_v, par)
            start_o(par, k)
        return 0

    lax.fori_loop(0, NCHUNK // 2, chunk_body, 0)
    wait_o(0)
    wait_o(1)


def _compute_chunk(tab_v, x_v, out_v, par):
        def row_body(r2, _):
            z = jnp.zeros((16,), jnp.float32)
            acc = [[z, z, z, z] for _ in range(4)]
            for i16 in range(IP // 16):
                vidx = [x_v[par, 4 * r2 + u, pl.ds(i16 * 16, 16)]
                        for u in range(4)]
                for l in range(0, min(16, I - i16 * 16), 2):
                    for u in range(4):
                        ps = tab_v[vidx[u][l]] + tab_v[vidx[u][l + 1]]
                        w = plsc.bitcast(ps, jnp.int32)
                        lo = plsc.bitcast(lax.shift_left(w, 16),
                                          jnp.float32)
                        hi = plsc.bitcast(lax.bitwise_and(w, _HIMASK),
                                          jnp.float32)
                        p = (l >> 1) & 1
                        acc[u][p] = acc[u][p] + lo
                        acc[u][2 + p] = acc[u][2 + p] + hi
            for u in range(4):
                out_v[par, 4 * r2 + u, pl.ds(0, 16)] = (
                    acc[u][0] + acc[u][1]
                )
                out_v[par, 4 * r2 + u, pl.ds(16, 16)] = (
                    acc[u][2] + acc[u][3]
                )
            return 0

        lax.fori_loop(0, RC // 4, row_body, 0)


@jax.jit
def _run(x_flat, tab2):
    mesh = plsc.VectorSubcoreMesh(core_axis_name="c", subcore_axis_name="s")
    f = pl.kernel(
        _sc_kernel,
        out_type=jax.ShapeDtypeStruct((NC, B, EH), jnp.float32),
        mesh=mesh,
        scratch_types=[
            pltpu.VMEM((TROWS, EH), jnp.bfloat16),
            pltpu.VMEM((2, RC, IP), jnp.int32),
            pltpu.VMEM((2, RC, EH), jnp.float32),
            pltpu.SemaphoreType.DMA,
            pltpu.SemaphoreType.DMA,
            pltpu.SemaphoreType.DMA,
            pltpu.SemaphoreType.DMA,
        ],
        compiler_params=pltpu.CompilerParams(
            use_tc_tiling_on_sc=False, needs_layout_passes=False
        ),
    )
    return f(x_flat, tab2)


_PERM = np.empty((EH,), np.int32)
_PERM[0::2] = np.arange(16)
_PERM[1::2] = np.arange(16) + 16


def kernel(x, embed_matrix):
    x = x.astype(jnp.int32)
    x_flat = x + (jnp.arange(I, dtype=jnp.int32) * V)[None, :]
    x_flat = jnp.concatenate(
        [x_flat, jnp.full((B, IP - I), I * V, jnp.int32)], axis=1
    )
    flat = embed_matrix.reshape(I * V, E)
    flat = jnp.concatenate(
        [flat, jnp.zeros((TROWS - I * V, E), jnp.float32)], axis=0
    )
    tab2 = flat.reshape(TROWS, NC, EH).transpose(1, 0, 2)
    tab2 = tab2[:, :, _PERM].astype(jnp.bfloat16)
    out3 = _run(x_flat, tab2)
    return out3.transpose(1, 0, 2).reshape(B, E)


# speedup vs baseline: 1.0010x; 1.0010x over previous
import jax
import jax.numpy as jnp
import numpy as np
from jax import lax
from jax.experimental import pallas as pl
from jax.experimental.pallas import tpu as pltpu
from jax.experimental.pallas import tpu_sc as plsc

B = 16384
I = 100
IP = 112
V = 31
E = 64
NC = 2
NS = 16
EH = E // NC
TROWS = I * V + 4
RPT = B // NS
RC = 128
NCHUNK = RPT // RC

_HIMASK = np.int32(-65536)


def _sc_kernel(x_hbm, tab_hbm, out_hbm, tab_v, x_v, out_v,
               sx0, sx1, so0, so1):
    sx = (sx0, sx1)
    so = (so0, so1)
    c = lax.axis_index("c")
    s = lax.axis_index("s")

    def start_x(buf, k):
        pltpu.async_copy(
            x_hbm.at[pl.ds(s * RPT + k * RC, RC), :], x_v.at[buf], sx[buf]
        )

    def wait_x(buf):
        pltpu.make_async_copy(
            x_hbm.at[pl.ds(0, RC), :], x_v.at[buf], sx[buf]
        ).wait()

    def start_o(buf, k):
        pltpu.async_copy(
            out_v.at[buf], out_hbm.at[c, pl.ds(s * RPT + k * RC, RC), :],
            so[buf],
        )

    def wait_o(buf):
        pltpu.make_async_copy(
            out_v.at[buf], out_hbm.at[c, pl.ds(0, RC), :], so[buf]
        ).wait()

    ht = pltpu.async_copy(tab_hbm.at[c], tab_v, sx[1])
    start_x(0, 0)
    ht.wait()

    def chunk_body(k2, _):
        for par in range(2):
            k = 2 * k2 + par

            @pl.when(k + 1 < NCHUNK)
            def _():
                start_x(1 - par, k + 1)

            wait_x(par)

            @pl.when(k >= 2)
            def _():
                wait_o(par)

            _compute_chunk(tab_v, x_v, out_v, par)
            start_o(par, k)
        return 0

    lax.fori_loop(0, NCHUNK // 2, chunk_body, 0)
    wait_o(0)
    wait_o(1)


def _compute_chunk(tab_v, x_v, out_v, par):
        def row_body(r2, _):
            z = jnp.zeros((16,), jnp.float32)
            acc = [[z, z, z, z] for _ in range(4)]
            for i16 in range(IP // 16):
                vidx = [x_v[par, 4 * r2 + u, pl.ds(i16 * 16, 16)]
                        for u in range(4)]
                for l in range(0, min(16, I - i16 * 16), 2):
                    for u in range(4):
                        ps = tab_v[vidx[u][l]] + tab_v[vidx[u][l + 1]]
                        w = plsc.bitcast(ps, jnp.int32)
                        lo = plsc.bitcast(lax.shift_left(w, 16),
                                          jnp.float32)
                        hi = plsc.bitcast(lax.bitwise_and(w, _HIMASK),
                                          jnp.float32)
                        p = (l >> 1) & 1
                        acc[u][p] = acc[u][p] + lo
                        acc[u][2 + p] = acc[u][2 + p] + hi
            for u in range(4):
                out_v[par, 4 * r2 + u, pl.ds(0, 16)] = (
                    acc[u][0] + acc[u][1]
                )
                out_v[par, 4 * r2 + u, pl.ds(16, 16)] = (
                    acc[u][2] + acc[u][3]
                )
            return 0

        lax.fori_loop(0, RC // 4, row_body, 0)


@jax.jit
def _run(x_flat, tab2):
    mesh = plsc.VectorSubcoreMesh(core_axis_name="c", subcore_axis_name="s")
    f = pl.kernel(
        _sc_kernel,
        out_type=jax.ShapeDtypeStruct((NC, B, EH), jnp.float32),
        mesh=mesh,
        scratch_types=[
            pltpu.VMEM((TROWS, EH), jnp.bfloat16),
            pltpu.VMEM((2, RC, IP), jnp.int32),
            pltpu.VMEM((2, RC, EH), jnp.float32),
            pltpu.SemaphoreType.DMA,
            pltpu.SemaphoreType.DMA,
            pltpu.SemaphoreType.DMA,
            pltpu.SemaphoreType.DMA,
        ],
        compiler_params=pltpu.CompilerParams(
            use_tc_tiling_on_sc=False, needs_layout_passes=False
        ),
    )
    return f(x_flat, tab2)


_PERM = np.empty((EH,), np.int32)
_PERM[0::2] = np.arange(16)
_PERM[1::2] = np.arange(16) + 16


def kernel(x, embed_matrix):
    x = x.astype(jnp.int32)
    x_flat = x + (jnp.arange(I, dtype=jnp.int32) * V)[None, :]
    x_flat = jnp.concatenate(
        [x_flat, jnp.full((B, IP - I), I * V, jnp.int32)], axis=1
    )
    flat = embed_matrix.reshape(I * V, E)
    flat = jnp.concatenate(
        [flat, jnp.zeros((TROWS - I * V, E), jnp.float32)], axis=0
    )
    tab2 = flat.reshape(TROWS, NC, EH).transpose(1, 0, 2)
    tab2 = tab2[:, :, _PERM].astype(jnp.bfloat16)
    out3 = _run(x_flat, tab2)
    return out3.transpose(1, 0, 2).reshape(B, E)
